# Initial kernel scaffold; baseline (speedup 1.0000x reference)
import jax
import jax.numpy as jnp
from jax.experimental import pallas as pl


def kernel(prefix_index, rest_index, fc_weight, bias):
    raise NotImplementedError("write your pallas kernel here")


# speedup vs baseline: 1.3070x; 1.3070x over previous
import functools

import jax
import jax.numpy as jnp
from jax import lax
from jax.experimental import pallas as pl
from jax.experimental.pallas import tpu as pltpu
from jax.experimental.pallas import tpu_sc as plsc

BATCH = 16384
NFIELD = 20
NPREFIX = 6
VOCAB = 40000
NC, NS, L = 2, 16, 16
NW = NC * NS
RPW = BATCH // NW
G = RPW * NFIELD
CHUNKS = G // L
RCHUNKS = RPW // L


def _body(prefix_hbm, rest_hbm, w_hbm, bias_hbm, out_hbm,
          idx_ref, vals_ref, acc_ref, sem):
    wid = lax.axis_index("s") * NC + lax.axis_index("c")
    base = wid * RPW

    pltpu.sync_copy(rest_hbm.at[pl.ds(wid * G, G)], idx_ref.at[pl.ds(0, G)])
    lanes = lax.iota(jnp.int32, L)
    idx_ref[pl.ds(G, L)] = jnp.zeros((L,), jnp.int32)

    pltpu.sync_copy(prefix_hbm, idx_ref.at[pl.ds(G, NPREFIX)])

    def add_off(c, f):
        idx_ref[pl.ds(c * L, L)] = idx_ref[pl.ds(c * L, L)] + (f + NPREFIX) * VOCAB
        fn = f + L
        return jnp.where(fn >= NFIELD, fn - NFIELD, fn)
    lax.fori_loop(0, CHUNKS, add_off, lanes)
    pv = idx_ref[pl.ds(G, L)]
    idx_ref[pl.ds(G, L)] = pv + jnp.where(lanes < NPREFIX, lanes * VOCAB, 0)

    pltpu.async_copy(w_hbm.at[idx_ref], vals_ref, sem).wait()

    acc_ref[pl.ds(0, L)] = jnp.zeros((L,), jnp.float32)
    pltpu.sync_copy(bias_hbm, acc_ref.at[pl.ds(0, 1)])
    bias_v = acc_ref[pl.ds(0, L)]
    pvals = vals_ref[pl.ds(G, L)]
    contrib = jnp.where(lanes < NPREFIX, pvals, 0.0) + bias_v
    sb_vec = contrib
    for k in (1, 2, 4, 8):
        sb_vec = sb_vec + sb_vec.at[lanes ^ k].get(mode="promise_in_bounds")

    def row_chunk(c, pos0):
        acc = sb_vec
        for f in range(NFIELD):
            acc = acc + plsc.load_gather(vals_ref, [pos0 + f])
        acc_ref[pl.ds(c * L, L)] = acc
        return pos0 + L * NFIELD
    lax.fori_loop(0, RCHUNKS, row_chunk, lanes * NFIELD)

    pltpu.sync_copy(acc_ref.at[pl.ds(0, RPW)], out_hbm.at[pl.ds(base, RPW)])


@jax.jit
def _run(prefix_index, rest_flat, w_flat, bias):
    mesh = plsc.VectorSubcoreMesh(core_axis_name="c", subcore_axis_name="s",
                                  num_cores=NC, num_subcores=NS)
    f = pl.kernel(
        _body,
        out_type=jax.ShapeDtypeStruct((BATCH,), jnp.float32),
        mesh=mesh,
        scratch_types=[
            pltpu.VMEM((G + L,), jnp.int32),
            pltpu.VMEM((G + L,), jnp.float32),
            pltpu.VMEM((RPW,), jnp.float32),
            pltpu.SemaphoreType.DMA,
        ],
        compiler_params=pltpu.CompilerParams(needs_layout_passes=False),
    )
    return f(prefix_index, rest_flat, w_flat, bias)


def kernel(prefix_index, rest_index, fc_weight, bias):
    rest_flat = rest_index.astype(jnp.int32).reshape(-1)
    out = _run(prefix_index.astype(jnp.int32), rest_flat,
               fc_weight.reshape(-1), bias)
    return out.reshape(BATCH, 1)


# speedup vs baseline: 1.3560x; 1.0374x over previous
import functools

import jax
import jax.numpy as jnp
from jax import lax
from jax.experimental import pallas as pl
from jax.experimental.pallas import tpu as pltpu
from jax.experimental.pallas import tpu_sc as plsc

BATCH = 16384
NFIELD = 20
NPREFIX = 6
VOCAB = 40000
NC, NS, L = 2, 16, 16
NW = NC * NS
RPW = BATCH // NW
G = RPW * NFIELD
RCHUNKS = RPW // L


def _body(prefix_hbm, rest_hbm, w_hbm, bias_hbm, out_hbm,
          stage_ref, idx_ref, vals_ref, acc_ref, pidx_ref, pval_ref, sem):
    wid = lax.axis_index("s") * NC + lax.axis_index("c")
    base = wid * RPW
    lanes = lax.iota(jnp.int32, L)

    pltpu.sync_copy(rest_hbm.at[pl.ds(wid * G, G)], stage_ref)

    pidx_ref[...] = jnp.zeros((L,), jnp.int32)
    pltpu.sync_copy(prefix_hbm, pidx_ref.at[pl.ds(0, NPREFIX)])
    pidx_ref[...] = pidx_ref[...] + jnp.where(lanes < NPREFIX,
                                              lanes * VOCAB, 0)
    pgather = pltpu.async_copy(w_hbm.at[pidx_ref], pval_ref, sem)

    gathers = []
    for f in range(NFIELD):
        def tr(c, pos, f=f):
            v = plsc.load_gather(stage_ref, [pos])
            idx_ref[pl.ds(f * RPW + c * L, L)] = v + (NPREFIX + f) * VOCAB
            return pos + L * NFIELD
        lax.fori_loop(0, RCHUNKS, tr, lanes * NFIELD + f)
        gathers.append(
            pltpu.async_copy(w_hbm.at[idx_ref.at[pl.ds(f * RPW, RPW)]],
                             vals_ref.at[pl.ds(f * RPW, RPW)], sem))

    pgather.wait()
    acc_ref[pl.ds(0, L)] = jnp.zeros((L,), jnp.float32)
    pltpu.sync_copy(bias_hbm, acc_ref.at[pl.ds(0, 1)])
    sb_vec = jnp.where(lanes < NPREFIX, pval_ref[...], 0.0) + acc_ref[pl.ds(0, L)]
    for k in (1, 2, 4, 8):
        sb_vec = sb_vec + sb_vec.at[lanes ^ k].get(mode="promise_in_bounds")

    for g in gathers:
        g.wait()

    def row_chunk(c, _):
        acc = sb_vec
        for f in range(NFIELD):
            acc = acc + vals_ref[pl.ds(f * RPW + c * L, L)]
        acc_ref[pl.ds(c * L, L)] = acc
        return _
    lax.fori_loop(0, RCHUNKS, row_chunk, 0)

    pltpu.sync_copy(acc_ref, out_hbm.at[pl.ds(base, RPW)])


@jax.jit
def _run(prefix_index, rest_flat, w_flat, bias):
    mesh = plsc.VectorSubcoreMesh(core_axis_name="c", subcore_axis_name="s",
                                  num_cores=NC, num_subcores=NS)
    f = pl.kernel(
        _body,
        out_type=jax.ShapeDtypeStruct((BATCH,), jnp.float32),
        mesh=mesh,
        scratch_types=[
            pltpu.VMEM((G,), jnp.int32),
            pltpu.VMEM((NFIELD * RPW,), jnp.int32),
            pltpu.VMEM((NFIELD * RPW,), jnp.float32),
            pltpu.VMEM((RPW,), jnp.float32),
            pltpu.VMEM((L,), jnp.int32),
            pltpu.VMEM((L,), jnp.float32),
            pltpu.SemaphoreType.DMA,
        ],
        compiler_params=pltpu.CompilerParams(needs_layout_passes=False),
    )
    return f(prefix_index, rest_flat, w_flat, bias)


def kernel(prefix_index, rest_index, fc_weight, bias):
    rest_flat = rest_index.astype(jnp.int32).reshape(-1)
    out = _run(prefix_index.astype(jnp.int32), rest_flat,
               fc_weight.reshape(-1), bias)
    return out.reshape(BATCH, 1)


# speedup vs baseline: 2.8538x; 2.1046x over previous
import functools

import jax
import jax.numpy as jnp
from jax import lax
from jax.experimental import pallas as pl
from jax.experimental.pallas import tpu as pltpu
from jax.experimental.pallas import tpu_sc as plsc

BATCH = 16384
NFIELD = 20
NPREFIX = 6
VOCAB = 40000
WPAD = 384
NC, NS, L = 2, 16, 16
NW = NC * NS
RPW = BATCH // NW
RCHUNKS = RPW // L


def _body(prefix_hbm, rest_hbm, w_hbm, bias_hbm, out_hbm,
          stage_ref, idx_ref, vals_ref, acc_ref, pidx_ref, pval_ref, sem):
    wid = lax.axis_index("s") * NC + lax.axis_index("c")
    base = wid * RPW
    lanes = lax.iota(jnp.int32, L)

    pltpu.sync_copy(rest_hbm.at[:, pl.ds(base, RPW)], stage_ref)

    pidx_ref[...] = jnp.zeros((L,), jnp.int32)
    pltpu.sync_copy(prefix_hbm, pidx_ref.at[pl.ds(0, NPREFIX)])
    pidx_ref[...] = pidx_ref[...] + jnp.where(lanes < NPREFIX,
                                              lanes * VOCAB, 0)
    pgather = pltpu.async_copy(w_hbm.at[pidx_ref], pval_ref, sem)

    gathers = []
    for f in range(NFIELD):
        def add_off(c, _, f=f):
            idx_ref[pl.ds(f * RPW + c * L, L)] = (
                stage_ref[f, pl.ds(c * L, L)] + (NPREFIX + f) * VOCAB)
            return _
        lax.fori_loop(0, RCHUNKS, add_off, 0)
        gathers.append(
            pltpu.async_copy(w_hbm.at[idx_ref.at[pl.ds(f * RPW, RPW)]],
                             vals_ref.at[pl.ds(f * RPW, RPW)], sem))

    pgather.wait()
    acc_ref[pl.ds(0, L)] = jnp.zeros((L,), jnp.float32)
    pltpu.sync_copy(bias_hbm, acc_ref.at[pl.ds(0, 1)])
    sb_vec = (jnp.where(lanes < NPREFIX, pval_ref[...], 0.0)
              + acc_ref[pl.ds(0, L)])
    for k in (1, 2, 4, 8):
        sb_vec = sb_vec + sb_vec.at[lanes ^ k].get(mode="promise_in_bounds")

    for g in gathers:
        g.wait()

    def row_chunk(c, _):
        acc = sb_vec
        for f in range(NFIELD):
            acc = acc + vals_ref[pl.ds(f * RPW + c * L, L)]
        acc_ref[pl.ds(c * L, L)] = acc
        return _
    lax.fori_loop(0, RCHUNKS, row_chunk, 0)

    pltpu.sync_copy(acc_ref, out_hbm.at[pl.ds(base, RPW)])


@jax.jit
def _run(prefix_index, rest_t, w_flat, bias):
    mesh = plsc.VectorSubcoreMesh(core_axis_name="c", subcore_axis_name="s",
                                  num_cores=NC, num_subcores=NS)
    f = pl.kernel(
        _body,
        out_type=jax.ShapeDtypeStruct((BATCH,), jnp.float32),
        mesh=mesh,
        scratch_types=[
            pltpu.VMEM((NFIELD, RPW), jnp.int32),
            pltpu.VMEM((NFIELD * RPW,), jnp.int32),
            pltpu.VMEM((NFIELD * RPW,), jnp.float32),
            pltpu.VMEM((RPW,), jnp.float32),
            pltpu.VMEM((L,), jnp.int32),
            pltpu.VMEM((L,), jnp.float32),
            pltpu.SemaphoreType.DMA,
        ],
        compiler_params=pltpu.CompilerParams(needs_layout_passes=False),
    )
    return f(prefix_index, rest_t, w_flat, bias)


def kernel(prefix_index, rest_index, fc_weight, bias):
    rest_t = rest_index.T
    w_flat = jnp.pad(fc_weight, ((0, WPAD), (0, 0))).reshape(-1)
    out = _run(prefix_index, rest_t, w_flat, bias)
    return out.reshape(BATCH, 1)


# speedup vs baseline: 2.8567x; 1.0010x over previous
import functools

import jax
import jax.numpy as jnp
from jax import lax
from jax.experimental import pallas as pl
from jax.experimental.pallas import tpu as pltpu
from jax.experimental.pallas import tpu_sc as plsc

BATCH = 16384
NFIELD = 20
NPREFIX = 6
VOCAB = 40000
WPAD = 384
NC, NS, L = 2, 16, 16
NW = NC * NS
RPW = BATCH // NW
RCHUNKS = RPW // L


def _body(prefix_hbm, rest_hbm, w_hbm, bias_hbm, out_hbm,
          stage_ref, idx_ref, vals_ref, acc_ref, pidx_ref, pval_ref, sem):
    wid = lax.axis_index("s") * NC + lax.axis_index("c")
    base = wid * RPW
    lanes = lax.iota(jnp.int32, L)

    with jax.named_scope("stage_idx"):
        pltpu.sync_copy(rest_hbm.at[:, pl.ds(base, RPW)], stage_ref)

    pidx_ref[...] = jnp.zeros((L,), jnp.int32)
    pltpu.sync_copy(prefix_hbm, pidx_ref.at[pl.ds(0, NPREFIX)])
    pidx_ref[...] = pidx_ref[...] + jnp.where(lanes < NPREFIX,
                                              lanes * VOCAB, 0)
    pgather = pltpu.async_copy(w_hbm.at[pidx_ref], pval_ref, sem)

    gathers = []
    with jax.named_scope("prep_fire"):
     for f in range(NFIELD):
        def add_off(c, _, f=f):
            idx_ref[pl.ds(f * RPW + c * L, L)] = (
                stage_ref[f, pl.ds(c * L, L)] + (NPREFIX + f) * VOCAB)
            return _
        lax.fori_loop(0, RCHUNKS, add_off, 0)
        gathers.append(
            pltpu.async_copy(w_hbm.at[idx_ref.at[pl.ds(f * RPW, RPW)]],
                             vals_ref.at[pl.ds(f * RPW, RPW)], sem))

    pgather.wait()
    acc_ref[pl.ds(0, L)] = jnp.zeros((L,), jnp.float32)
    pltpu.sync_copy(bias_hbm, acc_ref.at[pl.ds(0, 1)])
    sb_vec = (jnp.where(lanes < NPREFIX, pval_ref[...], 0.0)
              + acc_ref[pl.ds(0, L)])
    for k in (1, 2, 4, 8):
        sb_vec = sb_vec + sb_vec.at[lanes ^ k].get(mode="promise_in_bounds")

    with jax.named_scope("drain"):
        for g in gathers:
            g.wait()

    def row_chunk(c, _):
        acc = sb_vec
        for f in range(NFIELD):
            acc = acc + vals_ref[pl.ds(f * RPW + c * L, L)]
        acc_ref[pl.ds(c * L, L)] = acc
        return _
    with jax.named_scope("reduce"):
        lax.fori_loop(0, RCHUNKS, row_chunk, 0)

    pltpu.sync_copy(acc_ref, out_hbm.at[pl.ds(base, RPW)])


@jax.jit
def _run(prefix_index, rest_t, w_flat, bias):
    mesh = plsc.VectorSubcoreMesh(core_axis_name="c", subcore_axis_name="s",
                                  num_cores=NC, num_subcores=NS)
    f = pl.kernel(
        _body,
        out_type=jax.ShapeDtypeStruct((BATCH,), jnp.float32),
        mesh=mesh,
        scratch_types=[
            pltpu.VMEM((NFIELD, RPW), jnp.int32),
            pltpu.VMEM((NFIELD * RPW,), jnp.int32),
            pltpu.VMEM((NFIELD * RPW,), jnp.float32),
            pltpu.VMEM((RPW,), jnp.float32),
            pltpu.VMEM((L,), jnp.int32),
            pltpu.VMEM((L,), jnp.float32),
            pltpu.SemaphoreType.DMA,
        ],
        compiler_params=pltpu.CompilerParams(needs_layout_passes=False),
    )
    return f(prefix_index, rest_t, w_flat, bias)


def kernel(prefix_index, rest_index, fc_weight, bias):
    rest_t = rest_index.T
    w_flat = jnp.pad(fc_weight, ((0, WPAD), (0, 0))).reshape(-1)
    out = _run(prefix_index, rest_t, w_flat, bias)
    return out.reshape(BATCH, 1)
